# baseline (speedup 1.0000x reference)
import functools

import jax
import jax.numpy as jnp
from jax import lax
from jax.experimental import pallas as pl
from jax.experimental.pallas import tpu as pltpu
from jax.experimental.pallas import tpu_sc as plsc

VOCAB = 100000
D = 300
D_PAD = 304
B = 16384
CTX = 20
NEG = 20
TN = NEG + 1
TN_PAD = 24

NC = 2
NS = 16
NW = NC * NS
BPW = B // NW
G = 2
NG = BPW // G
SCORE_PAD = 32

NCH = D_PAD // 16


def _sc_scores(ctx_idx, tn_idx, w_in, w_out):
    mesh = plsc.VectorSubcoreMesh(
        core_axis_name="c", subcore_axis_name="s", num_cores=NC, num_subcores=NS
    )

    @functools.partial(
        pl.kernel,
        out_type=jax.ShapeDtypeStruct((NW, NG, G * SCORE_PAD), jnp.float32),
        mesh=mesh,
        compiler_params=pltpu.CompilerParams(
            needs_layout_passes=False, use_tc_tiling_on_sc=False
        ),
        scratch_types=[
            pltpu.VMEM((NG, G * CTX), jnp.int32),
            pltpu.VMEM((NG, G * TN_PAD), jnp.int32),
            pltpu.VMEM((G * CTX, D_PAD), jnp.float32),
            pltpu.VMEM((G * CTX, D_PAD), jnp.float32),
            pltpu.VMEM((G * TN_PAD, D_PAD), jnp.float32),
            pltpu.VMEM((G * TN_PAD, D_PAD), jnp.float32),
            pltpu.VMEM((NG, G * SCORE_PAD), jnp.float32),
            pltpu.SemaphoreType.DMA,
            pltpu.SemaphoreType.DMA,
            pltpu.SemaphoreType.DMA,
            pltpu.SemaphoreType.DMA,
        ],
    )
    def k(ctx_idx_hbm, tn_idx_hbm, w_in_hbm, w_out_hbm, out_hbm,
          ctx_idx_v, tn_idx_v, cb0, cb1, ob0, ob1, scores_v,
          sc0, sc1, so0, so1):
        cbufs, obufs = (cb0, cb1), (ob0, ob1)
        csems, osems = (sc0, sc1), (so0, so1)
        wid = lax.axis_index("s") * NC + lax.axis_index("c")
        pltpu.sync_copy(ctx_idx_hbm.at[wid], ctx_idx_v)
        pltpu.sync_copy(tn_idx_hbm.at[wid], tn_idx_v)

        lane = lax.iota(jnp.int32, 16)
        zero = jnp.zeros((16,), jnp.float32)
        inv_ctx = jnp.float32(1.0 / CTX)

        def ctx_accum(e, ctx_rows_v):
            def ctx_body(r, accs):
                row = e * CTX + r
                return tuple(
                    accs[j] + ctx_rows_v[row, pl.ds(j * 16, 16)]
                    for j in range(NCH)
                )

            accs = lax.fori_loop(0, CTX, ctx_body, (zero,) * NCH)
            return [a * inv_ctx for a in accs]

        def dots(g, e, ctxc, out_rows_v):
            def dot_body(r2, svecs):
                sv0, sv1 = svecs
                row = e * TN_PAD + r2
                acc = ctxc[0] * out_rows_v[row, pl.ds(0, 16)]
                for j in range(1, NCH):
                    acc = acc + ctxc[j] * out_rows_v[row, pl.ds(j * 16, 16)]
                s = jnp.sum(acc)
                s = jnp.where(r2 == 0, s, -s)
                sv0 = jnp.where(lane == r2, s, sv0)
                sv1 = jnp.where(lane == r2 - 16, s, sv1)
                return sv0, sv1

            sv0, sv1 = lax.fori_loop(0, TN, dot_body, (zero, zero))
            scores_v[g, pl.ds(e * SCORE_PAD, 16)] = sv0
            scores_v[g, pl.ds(e * SCORE_PAD + 16, 16)] = sv1

        def issue(g, b):
            pltpu.async_copy(w_in_hbm.at[ctx_idx_v.at[g]], cbufs[b], csems[b])
            pltpu.async_copy(w_out_hbm.at[tn_idx_v.at[g]], obufs[b], osems[b])

        def step(g, b, prefetch):
            pltpu.make_async_copy(
                w_in_hbm.at[ctx_idx_v.at[g]], cbufs[b], csems[b]).wait()
            pltpu.make_async_copy(
                w_out_hbm.at[tn_idx_v.at[g]], obufs[b], osems[b]).wait()
            ctxc0 = ctx_accum(0, cbufs[b])
            dots(g, 0, ctxc0, obufs[b])
            ctxc1 = ctx_accum(1, cbufs[b])
            dots(g, 1, ctxc1, obufs[b])
            if prefetch:
                issue(g + 2, b)

        issue(0, 0)
        issue(1, 1)

        def outer(i, carry):
            step(i * 2, 0, True)
            step(i * 2 + 1, 1, True)
            return carry

        lax.fori_loop(0, NG // 2 - 1, outer, 0)
        step(NG - 2, 0, False)
        step(NG - 1, 1, False)
        pltpu.sync_copy(scores_v, out_hbm.at[wid])

    return k(ctx_idx, tn_idx, w_in, w_out)


def _loss_body(x_ref, o_ref):
    x = x_ref[...]
    col = lax.broadcasted_iota(jnp.int32, x.shape, 1) % SCORE_PAD
    valid = col < TN
    ls = jnp.minimum(x, 0.0) - jnp.log1p(jnp.exp(-jnp.abs(x)))
    o_ref[0, 0] = -jnp.sum(jnp.where(valid, ls, 0.0)) * jnp.float32(1.0 / B)


def kernel(context, target, neg_samples, W_in, W_out):
    context = context.astype(jnp.int32)
    target = target.astype(jnp.int32)
    neg_samples = neg_samples.astype(jnp.int32)

    ctx_idx = context.reshape(NW, NG, G * CTX)
    tn = jnp.concatenate(
        [target[:, None], neg_samples,
         jnp.zeros((B, TN_PAD - TN), jnp.int32)], axis=1)
    tn_idx = tn.reshape(NW, NG, G * TN_PAD)

    w_in_p = jnp.pad(W_in, ((0, 0), (0, D_PAD - D)))
    w_out_p = jnp.pad(W_out, ((0, 0), (0, D_PAD - D)))

    scores = _sc_scores(ctx_idx, tn_idx, w_in_p, w_out_p)

    scores2d = scores.reshape(B * SCORE_PAD // 128, 128)
    loss = pl.pallas_call(
        _loss_body,
        out_shape=jax.ShapeDtypeStruct((1, 1), jnp.float32),
        out_specs=pl.BlockSpec(memory_space=pltpu.SMEM),
    )(scores2d)
    return loss[0, 0]


# speedup vs baseline: 1.1430x; 1.1430x over previous
import functools

import jax
import jax.numpy as jnp
from jax import lax
from jax.experimental import pallas as pl
from jax.experimental.pallas import tpu as pltpu
from jax.experimental.pallas import tpu_sc as plsc

VOCAB = 100000
D = 300
D_PAD = 304
B = 16384
CTX = 20
NEG = 20
TN = NEG + 1
TN_PAD = 24

NC = 2
NS = 16
NW = NC * NS
BPW = B // NW
G = 2
NG = BPW // G
SCORE_PAD = 32

NCH = D_PAD // 16


def _sc_scores(ctx_idx, tn_idx, w_in, w_out):
    mesh = plsc.VectorSubcoreMesh(
        core_axis_name="c", subcore_axis_name="s", num_cores=NC, num_subcores=NS
    )

    @functools.partial(
        pl.kernel,
        out_type=jax.ShapeDtypeStruct((NW, NG, G * SCORE_PAD), jnp.float32),
        mesh=mesh,
        compiler_params=pltpu.CompilerParams(
            needs_layout_passes=False, use_tc_tiling_on_sc=False
        ),
        scratch_types=[
            pltpu.VMEM((NG, G * CTX), jnp.int32),
            pltpu.VMEM((NG, G * TN_PAD), jnp.int32),
            pltpu.VMEM((G * CTX, D_PAD), jnp.float32),
            pltpu.VMEM((G * CTX, D_PAD), jnp.float32),
            pltpu.VMEM((G * TN_PAD, D_PAD), jnp.float32),
            pltpu.VMEM((G * TN_PAD, D_PAD), jnp.float32),
            pltpu.VMEM((NG, G * SCORE_PAD), jnp.float32),
            pltpu.SemaphoreType.DMA,
            pltpu.SemaphoreType.DMA,
            pltpu.SemaphoreType.DMA,
            pltpu.SemaphoreType.DMA,
        ],
    )
    def k(ctx_idx_hbm, tn_idx_hbm, w_in_hbm, w_out_hbm, out_hbm,
          ctx_idx_v, tn_idx_v, cb0, cb1, ob0, ob1, scores_v,
          sc0, sc1, so0, so1):
        cbufs, obufs = (cb0, cb1), (ob0, ob1)
        csems, osems = (sc0, sc1), (so0, so1)
        wid = lax.axis_index("s") * NC + lax.axis_index("c")
        pltpu.sync_copy(ctx_idx_hbm.at[wid], ctx_idx_v)
        pltpu.sync_copy(tn_idx_hbm.at[wid], tn_idx_v)

        lane = lax.iota(jnp.int32, 16)
        zero = jnp.zeros((16,), jnp.float32)
        inv_ctx = jnp.float32(1.0 / CTX)

        def ctx_accum(e, ctx_rows_v):
            def ctx_body(r, accs):
                row = e * CTX + r
                return tuple(
                    accs[j] + ctx_rows_v[row, pl.ds(j * 16, 16)]
                    for j in range(NCH)
                )

            accs = lax.fori_loop(0, CTX, ctx_body, (zero,) * NCH, unroll=4)
            return [a * inv_ctx for a in accs]

        def dots(g, e, ctxc, out_rows_v):
            def dot_body(r2, svecs):
                sv0, sv1 = svecs
                row = e * TN_PAD + r2
                parts = [ctxc[j] * out_rows_v[row, pl.ds(j * 16, 16)]
                         for j in range(NCH)]
                while len(parts) > 1:
                    parts = [parts[i] + parts[i + 1]
                             if i + 1 < len(parts) else parts[i]
                             for i in range(0, len(parts), 2)]
                s = jnp.sum(parts[0])
                s = jnp.where(r2 == 0, s, -s)
                sv0 = jnp.where(lane == r2, s, sv0)
                sv1 = jnp.where(lane == r2 - 16, s, sv1)
                return sv0, sv1

            sv0, sv1 = lax.fori_loop(0, TN, dot_body, (zero, zero), unroll=3)
            scores_v[g, pl.ds(e * SCORE_PAD, 16)] = sv0
            scores_v[g, pl.ds(e * SCORE_PAD + 16, 16)] = sv1

        def issue(g, b):
            pltpu.async_copy(w_in_hbm.at[ctx_idx_v.at[g]], cbufs[b], csems[b])
            pltpu.async_copy(w_out_hbm.at[tn_idx_v.at[g]], obufs[b], osems[b])

        def step(g, b, prefetch):
            pltpu.make_async_copy(
                w_in_hbm.at[ctx_idx_v.at[g]], cbufs[b], csems[b]).wait()
            pltpu.make_async_copy(
                w_out_hbm.at[tn_idx_v.at[g]], obufs[b], osems[b]).wait()
            ctxc0 = ctx_accum(0, cbufs[b])
            dots(g, 0, ctxc0, obufs[b])
            ctxc1 = ctx_accum(1, cbufs[b])
            dots(g, 1, ctxc1, obufs[b])
            if prefetch:
                issue(g + 2, b)

        issue(0, 0)
        issue(1, 1)

        def outer(i, carry):
            step(i * 2, 0, True)
            step(i * 2 + 1, 1, True)
            return carry

        lax.fori_loop(0, NG // 2 - 1, outer, 0)
        step(NG - 2, 0, False)
        step(NG - 1, 1, False)
        pltpu.sync_copy(scores_v, out_hbm.at[wid])

    return k(ctx_idx, tn_idx, w_in, w_out)


def _pad_body(x_ref, o_ref):
    o_ref[...] = jnp.concatenate(
        [x_ref[...], jnp.zeros((x_ref.shape[0], D_PAD - D), jnp.float32)],
        axis=1)


def _pad_table(w):
    rows = 2000
    return pl.pallas_call(
        _pad_body,
        grid=(VOCAB // rows,),
        in_specs=[pl.BlockSpec((rows, D), lambda i: (i, 0))],
        out_specs=pl.BlockSpec((rows, D_PAD), lambda i: (i, 0)),
        out_shape=jax.ShapeDtypeStruct((VOCAB, D_PAD), jnp.float32),
    )(w)


def _loss_body(x_ref, o_ref):
    x = x_ref[...]
    col = lax.broadcasted_iota(jnp.int32, x.shape, 1) % SCORE_PAD
    valid = col < TN
    ls = jnp.minimum(x, 0.0) - jnp.log1p(jnp.exp(-jnp.abs(x)))
    o_ref[0, 0] = -jnp.sum(jnp.where(valid, ls, 0.0)) * jnp.float32(1.0 / B)


def kernel(context, target, neg_samples, W_in, W_out):
    context = context.astype(jnp.int32)
    target = target.astype(jnp.int32)
    neg_samples = neg_samples.astype(jnp.int32)

    ctx_idx = context.reshape(NW, NG, G * CTX)
    tn = jnp.concatenate(
        [target[:, None], neg_samples,
         jnp.zeros((B, TN_PAD - TN), jnp.int32)], axis=1)
    tn_idx = tn.reshape(NW, NG, G * TN_PAD)

    w_in_p = _pad_table(W_in)
    w_out_p = _pad_table(W_out)

    scores = _sc_scores(ctx_idx, tn_idx, w_in_p, w_out_p)

    scores2d = scores.reshape(B * SCORE_PAD // 128, 128)
    loss = pl.pallas_call(
        _loss_body,
        out_shape=jax.ShapeDtypeStruct((1, 1), jnp.float32),
        out_specs=pl.BlockSpec(memory_space=pltpu.SMEM),
    )(scores2d)
    return loss[0, 0]
